# Initial kernel scaffold; baseline (speedup 1.0000x reference)
import jax
import jax.numpy as jnp
from jax.experimental import pallas as pl


def kernel(pos_u, pos_v, neg_v, u_table, v_table):
    raise NotImplementedError("write your pallas kernel here")


# speedup vs baseline: 5.0932x; 5.0932x over previous
import functools

import jax
import jax.numpy as jnp
from jax import lax
from jax.experimental import pallas as pl
from jax.experimental.pallas import tpu as pltpu
from jax.experimental.pallas import tpu_sc as plsc

VOCAB = 1000000
DIM = 64
BATCH = 16384
NNEG = 20

NC = 2
NS = 16
LANES = 16
NW = NC * NS
B_PER_W = BATCH // NW
GROUP = 64
NGROUP = B_PER_W // GROUP
NEG_CHUNK = 128
NEG_STREAMS = GROUP * NNEG // NEG_CHUNK


def _sc_body(pos_u_hbm, pos_v_hbm, neg_flat_hbm, u_table, v_table,
             pos_out, neg_out,
             pu_idx, pv_idx, ng_idx, u_rows, v_rows, n_rows,
             pos_s, neg_s, sem):
  wid = lax.axis_index("s") * NC + lax.axis_index("c")

  def group_body(g, carry):
    base = wid * B_PER_W + g * GROUP
    pltpu.sync_copy(pos_u_hbm.at[pl.ds(base, GROUP)], pu_idx)
    pltpu.sync_copy(pos_v_hbm.at[pl.ds(base, GROUP)], pv_idx)
    for j in range(NEG_STREAMS):
      pltpu.sync_copy(
          neg_flat_hbm.at[pl.ds(base * NNEG + j * NEG_CHUNK, NEG_CHUNK)],
          ng_idx.at[j])

    copies = [
        pltpu.async_copy(u_table.at[pu_idx], u_rows, sem),
        pltpu.async_copy(v_table.at[pv_idx], v_rows, sem),
    ]
    for j in range(NEG_STREAMS):
      copies.append(
          pltpu.async_copy(v_table.at[ng_idx.at[j]],
                           n_rows.at[pl.ds(j * NEG_CHUNK, NEG_CHUNK)], sem))
    for c in copies:
      c.wait()

    def elem_body(b, carry2):
      u = [u_rows[b, pl.ds(j * LANES, LANES)] for j in range(4)]
      p = u[0] * v_rows[b, pl.ds(0, LANES)]
      for j in range(1, 4):
        p = p + u[j] * v_rows[b, pl.ds(j * LANES, LANES)]
      nb = b * NNEG
      acc = [n_rows[nb, pl.ds(j * LANES, LANES)] for j in range(4)]
      for n in range(1, NNEG):
        for j in range(4):
          acc[j] = acc[j] + n_rows[nb + n, pl.ds(j * LANES, LANES)]
      q = acc[0] * u[0]
      for j in range(1, 4):
        q = q + acc[j] * u[j]
      pos_s[b, :] = p
      neg_s[b, :] = q
      return carry2

    lax.fori_loop(0, GROUP, elem_body, 0)

    pltpu.sync_copy(pos_s, pos_out.at[pl.ds(base, GROUP)])
    pltpu.sync_copy(neg_s, neg_out.at[pl.ds(base, GROUP)])
    return carry

  lax.fori_loop(0, NGROUP, group_body, 0)


@jax.jit
def _sc_scores(pos_u, pos_v, neg_flat, u_table, v_table):
  mesh = plsc.VectorSubcoreMesh(
      core_axis_name="c", subcore_axis_name="s",
      num_cores=NC, num_subcores=NS)
  return pl.kernel(
      _sc_body,
      out_type=(
          jax.ShapeDtypeStruct((BATCH, LANES), jnp.float32),
          jax.ShapeDtypeStruct((BATCH, LANES), jnp.float32),
      ),
      mesh=mesh,
      scratch_types=[
          pltpu.VMEM((GROUP,), jnp.int32),
          pltpu.VMEM((GROUP,), jnp.int32),
          pltpu.VMEM((NEG_STREAMS, NEG_CHUNK), jnp.int32),
          pltpu.VMEM((GROUP, DIM), jnp.float32),
          pltpu.VMEM((GROUP, DIM), jnp.float32),
          pltpu.VMEM((GROUP * NNEG, DIM), jnp.float32),
          pltpu.VMEM((GROUP, LANES), jnp.float32),
          pltpu.VMEM((GROUP, LANES), jnp.float32),
          pltpu.SemaphoreType.DMA,
      ],
      compiler_params=pltpu.CompilerParams(use_tc_tiling_on_sc=False),
      name="w2v_sc_gather_score",
  )(pos_u, pos_v, neg_flat, u_table, v_table)


TC_ROWS = 2048


def _tc_loss_body(p_ref, n_ref, o_ref):
  i = pl.program_id(0)

  @pl.when(i == 0)
  def _():
    o_ref[0, 0] = 0.0

  p = jnp.sum(p_ref[...], axis=1)
  n = -jnp.sum(n_ref[...], axis=1)
  lp = jnp.minimum(p, 0.0) - jnp.log1p(jnp.exp(-jnp.abs(p)))
  ln = jnp.minimum(n, 0.0) - jnp.log1p(jnp.exp(-jnp.abs(n)))
  o_ref[0, 0] += -(jnp.sum(lp) + jnp.sum(ln))


@jax.jit
def _tc_loss(pos_s, neg_s):
  out = pl.pallas_call(
      _tc_loss_body,
      grid=(BATCH // TC_ROWS,),
      in_specs=[
          pl.BlockSpec((TC_ROWS, LANES), lambda i: (i, 0)),
          pl.BlockSpec((TC_ROWS, LANES), lambda i: (i, 0)),
      ],
      out_shape=jax.ShapeDtypeStruct((1, 1), jnp.float32),
      out_specs=pl.BlockSpec(memory_space=pltpu.SMEM),
  )(pos_s, neg_s)
  return out[0, 0]


def kernel(pos_u, pos_v, neg_v, u_table, v_table):
  neg_flat = neg_v.reshape(-1)
  pos_s, neg_s = _sc_scores(pos_u, pos_v, neg_flat, u_table, v_table)
  return _tc_loss(pos_s, neg_s)
